# baseline (speedup 1.0000x reference)
import functools

import jax
import jax.numpy as jnp
from jax import lax
from jax.experimental import pallas as pl
from jax.experimental.pallas import tpu as pltpu
from jax.experimental.pallas import tpu_sc as plsc

B = 16384
D = 64
T = 384
NC = 2
NS = 16
NW = NC * NS
RPW = B // NW
CH = 128
NCH = RPW // CH

BLK = 512


@functools.cache
def _sc_gather():
    mesh = plsc.VectorSubcoreMesh(core_axis_name="c", subcore_axis_name="s")

    @functools.partial(
        pl.kernel,
        mesh=mesh,
        out_type=[
            jax.ShapeDtypeStruct((B, D), jnp.float32),
            jax.ShapeDtypeStruct((B, D), jnp.float32),
        ],
        scratch_types=[
            pltpu.VMEM((RPW,), jnp.int32),
            pltpu.VMEM((RPW,), jnp.int32),
            pltpu.VMEM((RPW, D), jnp.float32),
            pltpu.VMEM((RPW, D), jnp.float32),
            pltpu.SemaphoreType.DMA,
            pltpu.SemaphoreType.DMA,
        ],
        compiler_params=pltpu.CompilerParams(use_tc_tiling_on_sc=False),
    )
    def gather_kernel(uid_hbm, cid_hbm, utab_hbm, itab_hbm,
                      uout_hbm, cout_hbm,
                      uid_v, cid_v, u_v, c_v, semu, semc):
        wid = lax.axis_index("s") * NC + lax.axis_index("c")
        base = wid * RPW
        pltpu.sync_copy(uid_hbm.at[pl.ds(base, RPW)], uid_v)
        pltpu.sync_copy(cid_hbm.at[pl.ds(base, RPW)], cid_v)
        copies = []
        for k in range(NCH):
            sl = pl.ds(k * CH, CH)
            copies.append(
                pltpu.async_copy(utab_hbm.at[uid_v.at[sl]], u_v.at[sl], semu))
            copies.append(
                pltpu.async_copy(itab_hbm.at[cid_v.at[sl]], c_v.at[sl], semc))
        for cp in copies:
            cp.wait()
        pltpu.sync_copy(u_v, uout_hbm.at[pl.ds(base, RPW)])
        pltpu.sync_copy(c_v, cout_hbm.at[pl.ds(base, RPW)])

    return gather_kernel


def _tc_body(x_ref, w_ref, b_ref, u_ref, c_ref, o_ref):
    enc = jnp.dot(x_ref[...], w_ref[...], preferred_element_type=jnp.float32)
    enc = enc + b_ref[...]
    s = jnp.sum(u_ref[...] * (c_ref[...] + enc), axis=1, keepdims=True)
    o_ref[...] = 1.0 / (1.0 + jnp.exp(-s))


def _tc_fused(x, w, b2, u_rows, c_rows):
    return pl.pallas_call(
        _tc_body,
        grid=(B // BLK,),
        in_specs=[
            pl.BlockSpec((BLK, T), lambda i: (i, 0)),
            pl.BlockSpec((T, D), lambda i: (0, 0)),
            pl.BlockSpec((1, D), lambda i: (0, 0)),
            pl.BlockSpec((BLK, D), lambda i: (i, 0)),
            pl.BlockSpec((BLK, D), lambda i: (i, 0)),
        ],
        out_specs=pl.BlockSpec((BLK, 1), lambda i: (i, 0)),
        out_shape=jax.ShapeDtypeStruct((B, 1), jnp.float32),
    )(x, w, b2, u_rows, c_rows)


def kernel(user_ids, content_ids, encoded_text, user_table, item_table,
           proj_W, proj_b):
    uid = user_ids.astype(jnp.int32)
    cid = content_ids.astype(jnp.int32)
    u_rows, c_rows = _sc_gather()(uid, cid, user_table, item_table)
    return _tc_fused(encoded_text, proj_W, proj_b.reshape(1, D),
                     u_rows, c_rows)
